# baseline (speedup 1.0000x reference)
import functools

import jax
import jax.numpy as jnp
from jax import lax
from jax.experimental import pallas as pl
from jax.experimental.pallas import tpu as pltpu

_SLOPE = 0.01


def _leaky(v):
    return jnp.where(v > 0, v, _SLOPE * v)


def _lrot(v, d):
    if d == 0:
        return v
    return jnp.concatenate([v[..., d:], v[..., :d]], axis=-1)


def _conv5(x, wk, b, r_out):
    xs = jnp.concatenate([x[:, dy:dy + r_out, :] for dy in range(5)], axis=0)
    acc = b
    for dx in range(5):
        p = jnp.einsum("ok,krw->orw", wk[dx], xs,
                       preferred_element_type=jnp.float32)
        acc = acc + _lrot(p, dx)
    return acc


def _convnet_kernel(x_ref, wk1_ref, b1_ref, wk2_ref, b2_ref, wk3_ref, b3_ref,
                    o_ref, *, nb):
    W = 87
    HP = 88
    x = x_ref[...].reshape(1, nb * HP, W)

    a = _conv5(x, wk1_ref[...], b1_ref[...], nb * HP - 4)
    a = _conv5(a, wk2_ref[...], b2_ref[...], nb * HP - 8)
    a = _conv5(a, wk3_ref[...], b3_ref[...], nb * HP - 12)

    cm = a
    for j in range(1, 5):
        cm = jnp.maximum(cm, _lrot(a, j))
    r4 = nb * HP - 16
    rm = cm[:, 0:r4, :]
    for i in range(1, 5):
        rm = jnp.maximum(rm, cm[:, i:i + r4, :])
    row = lax.broadcasted_iota(jnp.int32, (r4, W), 0)
    col = lax.broadcasted_iota(jnp.int32, (r4, W), 1)
    ok = jnp.logical_and(row % HP <= 70, col <= 70)
    rm = jnp.where(ok[None, :, :], rm, 0.0)

    cw = lax.broadcasted_iota(jnp.int32, (W, 15), 0)
    vw = lax.broadcasted_iota(jnp.int32, (W, 15), 1)
    sw = jnp.where(cw == 5 * vw, 1.0, 0.0)
    pw = jnp.einsum("crw,wv->crv", rm, sw,
                    preferred_element_type=jnp.float32)

    u = nb * 15
    iu = lax.broadcasted_iota(jnp.int32, (u, r4), 0)
    ir = lax.broadcasted_iota(jnp.int32, (u, r4), 1)
    sr = jnp.where(ir == HP * (iu // 15) + 5 * (iu % 15), 1.0, 0.0)
    f3 = jnp.einsum("ur,crv->cuv", sr, pw,
                    preferred_element_type=jnp.float32)

    o_ref[...] = _leaky(f3)


def _fc1_kernel(x_ref, w_ref, b_ref, o_ref):
    y = lax.dot_general(x_ref[...], w_ref[...], (((1,), (1,)), ((), ())),
                        preferred_element_type=jnp.float32)
    o_ref[...] = _leaky(y + b_ref[...])


def _fc23_kernel(h_ref, w2_ref, b2_ref, w3_ref, b3_ref, o_ref):
    h2 = lax.dot_general(h_ref[...], w2_ref[...], (((1,), (1,)), ((), ())),
                         preferred_element_type=jnp.float32)
    h2 = _leaky(h2 + b2_ref[...])
    y = lax.dot_general(h2, w3_ref[...], (((1,), (1,)), ((), ())),
                        preferred_element_type=jnp.float32)
    o_ref[...] = y + b3_ref[...]


def _prep_w(w):
    o, c = w.shape[0], w.shape[1]
    return jnp.transpose(w, (3, 0, 2, 1)).reshape(5, o, 5 * c)


def _prep_b(b):
    return jnp.broadcast_to(b.reshape(-1, 1, 1), (b.shape[0], 1, 87))


def kernel(conv1_w, conv1_b, conv2_w, conv2_b, conv3_w, conv3_b,
           fc1_w, fc1_b, fc2_w, fc2_b, fc3_w, fc3_b, x):
    n = x.shape[0]
    nb = 8
    xp = jnp.pad(x.reshape(n, 87, 87), ((0, 0), (0, 1), (0, 0)))

    whole3 = lambda i: (0, 0, 0)
    fr = pl.pallas_call(
        functools.partial(_convnet_kernel, nb=nb),
        out_shape=jax.ShapeDtypeStruct((15, n * 15, 15), jnp.float32),
        grid=(n // nb,),
        in_specs=[
            pl.BlockSpec((nb, 88, 87), lambda i: (i, 0, 0)),
            pl.BlockSpec((5, 5, 5), whole3),
            pl.BlockSpec((5, 1, 87), whole3),
            pl.BlockSpec((5, 10, 25), whole3),
            pl.BlockSpec((10, 1, 87), whole3),
            pl.BlockSpec((5, 15, 50), whole3),
            pl.BlockSpec((15, 1, 87), whole3),
        ],
        out_specs=pl.BlockSpec((15, nb * 15, 15), lambda i: (0, i, 0)),
        compiler_params=pltpu.CompilerParams(
            dimension_semantics=("parallel",)),
    )(xp, _prep_w(conv1_w), _prep_b(conv1_b),
      _prep_w(conv2_w), _prep_b(conv2_b),
      _prep_w(conv3_w), _prep_b(conv3_b))
    f = fr.reshape(15, n, 15, 15).transpose(1, 0, 2, 3).reshape(n, 3375)

    tn = 512
    h1 = pl.pallas_call(
        _fc1_kernel,
        out_shape=jax.ShapeDtypeStruct((n, 2048), jnp.float32),
        grid=(2048 // tn,),
        in_specs=[
            pl.BlockSpec((n, 3375), lambda j: (0, 0)),
            pl.BlockSpec((tn, 3375), lambda j: (j, 0)),
            pl.BlockSpec((1, tn), lambda j: (0, j)),
        ],
        out_specs=pl.BlockSpec((n, tn), lambda j: (0, j)),
        compiler_params=pltpu.CompilerParams(
            dimension_semantics=("parallel",)),
    )(f, fc1_w, fc1_b.reshape(1, 2048))

    return pl.pallas_call(
        _fc23_kernel,
        out_shape=jax.ShapeDtypeStruct((n, 405), jnp.float32),
        in_specs=[
            pl.BlockSpec((n, 2048), lambda: (0, 0)),
            pl.BlockSpec((1024, 2048), lambda: (0, 0)),
            pl.BlockSpec((1, 1024), lambda: (0, 0)),
            pl.BlockSpec((405, 1024), lambda: (0, 0)),
            pl.BlockSpec((1, 405), lambda: (0, 0)),
        ],
        out_specs=pl.BlockSpec((n, 405), lambda: (0, 0)),
    )(h1, fc2_w, fc2_b.reshape(1, 1024), fc3_w, fc3_b.reshape(1, 405))


# speedup vs baseline: 2.0300x; 2.0300x over previous
import functools

import jax
import jax.numpy as jnp
from jax import lax
from jax.experimental import pallas as pl
from jax.experimental.pallas import tpu as pltpu

_SLOPE = 0.01


def _leaky(v):
    return jnp.where(v > 0, v, _SLOPE * v)


def _lrot(v, d):
    if d == 0:
        return v
    return jnp.concatenate([v[..., d:], v[..., :d]], axis=-1)


def _conv5(x, wk, b, r_out):
    xs = jnp.concatenate([x[:, dy:dy + r_out, :] for dy in range(5)], axis=0)
    acc = b
    for dx in range(5):
        p = jnp.einsum("ok,krw->orw", wk[dx], xs,
                       preferred_element_type=jnp.float32)
        acc = acc + _lrot(p, dx)
    return acc


def _convnet_kernel(x_ref, wk1_ref, b1_ref, wk2_ref, b2_ref, wk3_ref, b3_ref,
                    o_ref, *, nb):
    W = 128
    HP = 88
    x = x_ref[...].reshape(1, nb * HP, W)

    a = _conv5(x, wk1_ref[...], b1_ref[...], nb * HP - 4)
    a = _conv5(a, wk2_ref[...], b2_ref[...], nb * HP - 8)
    a = _conv5(a, wk3_ref[...], b3_ref[...], nb * HP - 12)

    cm = a
    for j in range(1, 5):
        cm = jnp.maximum(cm, _lrot(a, j))
    r4 = nb * HP - 16
    rm = cm[:, 0:r4, :]
    for i in range(1, 5):
        rm = jnp.maximum(rm, cm[:, i:i + r4, :])
    row = lax.broadcasted_iota(jnp.int32, (r4, W), 0)
    col = lax.broadcasted_iota(jnp.int32, (r4, W), 1)
    ok = jnp.logical_and(row % HP <= 70, col <= 70)
    rm = jnp.where(ok[None, :, :], rm, 0.0)

    cw = lax.broadcasted_iota(jnp.int32, (W, 15), 0)
    vw = lax.broadcasted_iota(jnp.int32, (W, 15), 1)
    sw = jnp.where(cw == 5 * vw, 1.0, 0.0)
    pw = jnp.einsum("crw,wv->crv", rm, sw,
                    preferred_element_type=jnp.float32)

    u = nb * 15
    iu = lax.broadcasted_iota(jnp.int32, (u, r4), 0)
    ir = lax.broadcasted_iota(jnp.int32, (u, r4), 1)
    sr = jnp.where(ir == HP * (iu // 15) + 5 * (iu % 15), 1.0, 0.0)
    f3 = jnp.einsum("ur,crv->cuv", sr, pw,
                    preferred_element_type=jnp.float32)

    f = f3.reshape(15, nb, 15, 15).transpose(1, 0, 2, 3).reshape(nb, 15 * 225)
    o_ref[...] = _leaky(f)


def _fc1_kernel(x_ref, w_ref, b_ref, o_ref):
    y = lax.dot_general(x_ref[...], w_ref[...], (((1,), (1,)), ((), ())),
                        preferred_element_type=jnp.float32)
    o_ref[...] = _leaky(y + b_ref[...])


def _fc23_kernel(h_ref, w2_ref, b2_ref, w3_ref, b3_ref, o_ref):
    h2 = lax.dot_general(h_ref[...], w2_ref[...], (((1,), (1,)), ((), ())),
                         preferred_element_type=jnp.float32)
    h2 = _leaky(h2 + b2_ref[...])
    y = lax.dot_general(h2, w3_ref[...], (((1,), (1,)), ((), ())),
                        preferred_element_type=jnp.float32)
    o_ref[...] = y + b3_ref[...]


def _prep_w(w):
    o, c = w.shape[0], w.shape[1]
    return jnp.transpose(w, (3, 0, 2, 1)).reshape(5, o, 5 * c)


def _prep_b(b):
    return jnp.broadcast_to(b.reshape(-1, 1, 1), (b.shape[0], 1, 128))


def kernel(conv1_w, conv1_b, conv2_w, conv2_b, conv3_w, conv3_b,
           fc1_w, fc1_b, fc2_w, fc2_b, fc3_w, fc3_b, x):
    n = x.shape[0]
    nb = 8
    xp = jnp.pad(x.reshape(n, 87, 87), ((0, 0), (0, 1), (0, 41)))

    whole3 = lambda i: (0, 0, 0)
    f = pl.pallas_call(
        functools.partial(_convnet_kernel, nb=nb),
        out_shape=jax.ShapeDtypeStruct((n, 3375), jnp.float32),
        grid=(n // nb,),
        in_specs=[
            pl.BlockSpec((nb, 88, 128), lambda i: (i, 0, 0)),
            pl.BlockSpec((5, 5, 5), whole3),
            pl.BlockSpec((5, 1, 128), whole3),
            pl.BlockSpec((5, 10, 25), whole3),
            pl.BlockSpec((10, 1, 128), whole3),
            pl.BlockSpec((5, 15, 50), whole3),
            pl.BlockSpec((15, 1, 128), whole3),
        ],
        out_specs=pl.BlockSpec((nb, 3375), lambda i: (i, 0)),
        compiler_params=pltpu.CompilerParams(
            dimension_semantics=("parallel",)),
    )(xp, _prep_w(conv1_w), _prep_b(conv1_b),
      _prep_w(conv2_w), _prep_b(conv2_b),
      _prep_w(conv3_w), _prep_b(conv3_b))

    tn = 512
    h1 = pl.pallas_call(
        _fc1_kernel,
        out_shape=jax.ShapeDtypeStruct((n, 2048), jnp.float32),
        grid=(2048 // tn,),
        in_specs=[
            pl.BlockSpec((n, 3375), lambda j: (0, 0)),
            pl.BlockSpec((tn, 3375), lambda j: (j, 0)),
            pl.BlockSpec((1, tn), lambda j: (0, j)),
        ],
        out_specs=pl.BlockSpec((n, tn), lambda j: (0, j)),
        compiler_params=pltpu.CompilerParams(
            dimension_semantics=("parallel",)),
    )(f, fc1_w, fc1_b.reshape(1, 2048))

    return pl.pallas_call(
        _fc23_kernel,
        out_shape=jax.ShapeDtypeStruct((n, 405), jnp.float32),
        in_specs=[
            pl.BlockSpec((n, 2048), lambda: (0, 0)),
            pl.BlockSpec((1024, 2048), lambda: (0, 0)),
            pl.BlockSpec((1, 1024), lambda: (0, 0)),
            pl.BlockSpec((405, 1024), lambda: (0, 0)),
            pl.BlockSpec((1, 405), lambda: (0, 0)),
        ],
        out_specs=pl.BlockSpec((n, 405), lambda: (0, 0)),
    )(h1, fc2_w, fc2_b.reshape(1, 1024), fc3_w, fc3_b.reshape(1, 405))


# speedup vs baseline: 2.6757x; 1.3181x over previous
import functools

import jax
import jax.numpy as jnp
from jax import lax
from jax.experimental import pallas as pl
from jax.experimental.pallas import tpu as pltpu

_SLOPE = 0.01


def _leaky(v):
    return jnp.where(v > 0, v, _SLOPE * v)


def _lrot(v, d):
    if d == 0:
        return v
    return jnp.concatenate([v[..., d:], v[..., :d]], axis=-1)


def _conv5(x, wk, b, rots, r_out):
    o = b.shape[0]
    xs = jnp.concatenate([x[:, dy:dy + r_out, :] for dy in range(5)], axis=0)
    p = jnp.einsum("ok,krw->orw", wk, xs,
                   preferred_element_type=jnp.float32)
    acc = b + p[0:o]
    for dx in range(1, 5):
        acc = acc + jnp.einsum("orw,wv->orv", p[dx * o:(dx + 1) * o],
                               rots[dx - 1],
                               preferred_element_type=jnp.float32)
    return acc


def _convnet_kernel(x_ref, wk1_ref, b1_ref, wk2_ref, b2_ref, wk3_ref, b3_ref,
                    o_ref, *, nb):
    W = 128
    HP = 88
    x = x_ref[...].reshape(1, nb * HP, W)

    rw = lax.broadcasted_iota(jnp.int32, (W, W), 0)
    rv = lax.broadcasted_iota(jnp.int32, (W, W), 1)
    rots = [jnp.where(rw == rv + d, 1.0, 0.0) for d in range(1, 5)]

    a = _conv5(x, wk1_ref[...], b1_ref[...], rots, nb * HP - 4)
    a = _conv5(a, wk2_ref[...], b2_ref[...], rots, nb * HP - 8)
    a = _conv5(a, wk3_ref[...], b3_ref[...], rots, nb * HP - 12)

    r4 = nb * HP - 16
    rm = a[:, 0:r4, :]
    for i in range(1, 5):
        rm = jnp.maximum(rm, a[:, i:i + r4, :])
    row = lax.broadcasted_iota(jnp.int32, (r4, W), 0)
    rm = jnp.where((row % HP <= 70)[None, :, :], rm, 0.0)

    u = nb * 15
    iu = lax.broadcasted_iota(jnp.int32, (u, r4), 0)
    ir = lax.broadcasted_iota(jnp.int32, (u, r4), 1)
    sr = jnp.where(ir == HP * (iu // 15) + 5 * (iu % 15), 1.0, 0.0)
    ps = jnp.einsum("ur,crw->cuw", sr, rm,
                    preferred_element_type=jnp.float32)

    cm = ps
    for j in range(1, 5):
        cm = jnp.maximum(cm, _lrot(ps, j))
    col = lax.broadcasted_iota(jnp.int32, (u, W), 1)
    cm = jnp.where((col <= 70)[None, :, :], cm, 0.0)

    cw = lax.broadcasted_iota(jnp.int32, (W, 15), 0)
    vw = lax.broadcasted_iota(jnp.int32, (W, 15), 1)
    sw = jnp.where(cw == 5 * vw, 1.0, 0.0)
    f3 = jnp.einsum("cuw,wv->cuv", cm, sw,
                    preferred_element_type=jnp.float32)

    f = f3.reshape(15, nb, 15, 15).transpose(1, 0, 2, 3).reshape(nb, 15 * 225)
    o_ref[...] = _leaky(f)


def _fc1_kernel(x_ref, w_ref, b_ref, o_ref):
    y = lax.dot_general(x_ref[...], w_ref[...], (((1,), (1,)), ((), ())),
                        preferred_element_type=jnp.float32)
    o_ref[...] = _leaky(y + b_ref[...])


def _fc23_kernel(h_ref, w2_ref, b2_ref, w3_ref, b3_ref, o_ref):
    h2 = lax.dot_general(h_ref[...], w2_ref[...], (((1,), (1,)), ((), ())),
                         preferred_element_type=jnp.float32)
    h2 = _leaky(h2 + b2_ref[...])
    y = lax.dot_general(h2, w3_ref[...], (((1,), (1,)), ((), ())),
                        preferred_element_type=jnp.float32)
    o_ref[...] = y + b3_ref[...]


def _prep_w(w):
    o, c = w.shape[0], w.shape[1]
    return jnp.transpose(w, (3, 0, 2, 1)).reshape(5 * o, 5 * c)


def _prep_b(b):
    return jnp.broadcast_to(b.reshape(-1, 1, 1), (b.shape[0], 1, 128))


def kernel(conv1_w, conv1_b, conv2_w, conv2_b, conv3_w, conv3_b,
           fc1_w, fc1_b, fc2_w, fc2_b, fc3_w, fc3_b, x):
    n = x.shape[0]
    nb = 8
    xp = jnp.pad(x.reshape(n, 87, 87), ((0, 0), (0, 1), (0, 41)))

    whole3 = lambda i: (0, 0, 0)
    f = pl.pallas_call(
        functools.partial(_convnet_kernel, nb=nb),
        out_shape=jax.ShapeDtypeStruct((n, 3375), jnp.float32),
        grid=(n // nb,),
        in_specs=[
            pl.BlockSpec((nb, 88, 128), lambda i: (i, 0, 0)),
            pl.BlockSpec((25, 5), lambda i: (0, 0)),
            pl.BlockSpec((5, 1, 128), whole3),
            pl.BlockSpec((50, 25), lambda i: (0, 0)),
            pl.BlockSpec((10, 1, 128), whole3),
            pl.BlockSpec((75, 50), lambda i: (0, 0)),
            pl.BlockSpec((15, 1, 128), whole3),
        ],
        out_specs=pl.BlockSpec((nb, 3375), lambda i: (i, 0)),
        compiler_params=pltpu.CompilerParams(
            dimension_semantics=("parallel",)),
    )(xp, _prep_w(conv1_w), _prep_b(conv1_b),
      _prep_w(conv2_w), _prep_b(conv2_b),
      _prep_w(conv3_w), _prep_b(conv3_b))

    tn = 512
    h1 = pl.pallas_call(
        _fc1_kernel,
        out_shape=jax.ShapeDtypeStruct((n, 2048), jnp.float32),
        grid=(2048 // tn,),
        in_specs=[
            pl.BlockSpec((n, 3375), lambda j: (0, 0)),
            pl.BlockSpec((tn, 3375), lambda j: (j, 0)),
            pl.BlockSpec((1, tn), lambda j: (0, j)),
        ],
        out_specs=pl.BlockSpec((n, tn), lambda j: (0, j)),
        compiler_params=pltpu.CompilerParams(
            dimension_semantics=("parallel",)),
    )(f, fc1_w, fc1_b.reshape(1, 2048))

    return pl.pallas_call(
        _fc23_kernel,
        out_shape=jax.ShapeDtypeStruct((n, 405), jnp.float32),
        in_specs=[
            pl.BlockSpec((n, 2048), lambda: (0, 0)),
            pl.BlockSpec((1024, 2048), lambda: (0, 0)),
            pl.BlockSpec((1, 1024), lambda: (0, 0)),
            pl.BlockSpec((405, 1024), lambda: (0, 0)),
            pl.BlockSpec((1, 405), lambda: (0, 0)),
        ],
        out_specs=pl.BlockSpec((n, 405), lambda: (0, 0)),
    )(h1, fc2_w, fc2_b.reshape(1, 1024), fc3_w, fc3_b.reshape(1, 405))
